# baseline (speedup 1.0000x reference)
import functools

import jax
import jax.numpy as jnp
import numpy as np
from jax import lax
from jax.experimental import pallas as pl
from jax.experimental.pallas import tpu as pltpu
from jax.experimental.pallas import tpu_sc as plsc

_MD = 11
_G = 24
_NT = 19
_ED = 8
_IN = _G * _MD
_OUT = _G * (_MD - 1 + _ED)
_NU = _OUT // 16
_NV = _G // 2


def _build_cmaps():
    rows = []
    masks = []
    for u in range(_NU):
        row = []
        mrow = []
        for lane in range(16):
            q = u * 16 + lane
            g, r = divmod(q, 18)
            row.append(g * _MD + 1 + r if r < 10 else g * _MD)
            mrow.append(1 if r < 10 else 0)
        rows.append(row)
        masks.append(mrow)
    rows.extend(masks)
    for v in range(_NV):
        rows.append([(2 * v + (1 if lane >= 8 else 0)) * _MD
                     for lane in range(16)])
    for v in range(_NV):
        rows.append([(2 * v + (1 if lane >= 8 else 0)) * 18 + 10 + (lane % 8)
                     for lane in range(16)])
    rows.append([lane % 8 for lane in range(16)])
    return np.array(rows, np.int32)


_CMAPS = _build_cmaps()
_ROW_MAP = 0
_ROW_PMSK = _NU
_ROW_CPOS = 2 * _NU
_ROW_OPOS = 2 * _NU + _NV
_ROW_CPAT = 2 * _NU + 2 * _NV


@jax.jit
def kernel(x, type_embedding):
    b, s = x.shape[0], x.shape[1]
    n = b * s
    xflat = x.reshape(n * _IN)
    tab = type_embedding.reshape(_NT * _ED)

    info = plsc.get_sparse_core_info()
    nw = info.num_cores * info.num_subcores
    rows_pw = n // nw
    chunk = 128
    while rows_pw % chunk != 0:
        chunk //= 2
    n_chunks = rows_pw // chunk

    mesh = plsc.VectorSubcoreMesh(core_axis_name="c", subcore_axis_name="s")

    @functools.partial(
        pl.kernel,
        mesh=mesh,
        compiler_params=pltpu.CompilerParams(needs_layout_passes=False),
        out_type=jax.ShapeDtypeStruct((n * _OUT,), jnp.float32),
        scratch_types=[
            pltpu.VMEM((_CMAPS.size,), jnp.int32),
            pltpu.VMEM((_NT * _ED,), jnp.float32),
            pltpu.VMEM((chunk * _IN,), jnp.float32),
            pltpu.VMEM((chunk * _OUT,), jnp.float32),
            pltpu.SemaphoreType.DMA,
            pltpu.SemaphoreType.DMA,
        ],
    )
    def k(x_hbm, tab_hbm, cm_hbm, out_hbm, cm_v, tab_v, in_v, out_v,
          sem_in, sem_out):
        wid = lax.axis_index("s") * info.num_cores + lax.axis_index("c")
        base = wid * rows_pw
        pltpu.sync_copy(cm_hbm, cm_v)
        pltpu.sync_copy(tab_hbm, tab_v)
        lanes = lax.iota(jnp.int32, 16)

        def chunk_body(ci, carry):
            r0 = base + ci * chunk
            pltpu.async_copy(
                x_hbm.at[pl.ds(r0 * _IN, chunk * _IN)], in_v, sem_in
            ).wait()

            for u in range(_NU):
                m = cm_v[pl.ds((_ROW_MAP + u) * 16, 16)]
                pm = cm_v[pl.ds((_ROW_PMSK + u) * 16, 16)] != 0
                oidx0 = lanes + u * 16

                @plsc.parallel_loop(0, chunk, unroll=4)
                def pbody(i, m=m, pm=pm, oidx0=oidx0):
                    vals = plsc.load_gather(in_v, [m + i * _IN])
                    plsc.store_scatter(
                        out_v, [oidx0 + i * _OUT], vals, mask=pm
                    )

            cpat = cm_v[pl.ds(_ROW_CPAT * 16, 16)]
            for v in range(_NV):
                cpos = cm_v[pl.ds((_ROW_CPOS + v) * 16, 16)]
                opos = cm_v[pl.ds((_ROW_OPOS + v) * 16, 16)]

                @plsc.parallel_loop(0, chunk, unroll=4)
                def ebody(i, cpos=cpos, opos=opos):
                    ch0 = plsc.load_gather(in_v, [cpos + i * _IN])
                    ei = ch0.astype(jnp.int32) * _ED + cpat
                    ev = plsc.load_gather(tab_v, [ei])
                    plsc.store_scatter(out_v, [opos + i * _OUT], ev)

            pltpu.async_copy(
                out_v, out_hbm.at[pl.ds(r0 * _OUT, chunk * _OUT)], sem_out
            ).wait()
            return carry

        lax.fori_loop(0, n_chunks, chunk_body, 0)

    out = k(xflat, tab, jnp.asarray(_CMAPS.reshape(-1)))
    return out.reshape(b, s, _OUT)


# speedup vs baseline: 1.0555x; 1.0555x over previous
import functools

import jax
import jax.numpy as jnp
import numpy as np
from jax import lax
from jax.experimental import pallas as pl
from jax.experimental.pallas import tpu as pltpu
from jax.experimental.pallas import tpu_sc as plsc

_MD = 11
_G = 24
_NT = 19
_ED = 8
_IN = _G * _MD
_OUT = _G * (_MD - 1 + _ED)
_NU = _OUT // 16
_NV = _G // 2


def _build_cmaps():
    rows = []
    masks = []
    for u in range(_NU):
        row = []
        mrow = []
        for lane in range(16):
            q = u * 16 + lane
            g, r = divmod(q, 18)
            row.append(g * _MD + 1 + r if r < 10 else g * _MD)
            mrow.append(1 if r < 10 else 0)
        rows.append(row)
        masks.append(mrow)
    rows.extend(masks)
    for v in range(_NV):
        rows.append([(2 * v + (1 if lane >= 8 else 0)) * _MD
                     for lane in range(16)])
    for v in range(_NV):
        rows.append([(2 * v + (1 if lane >= 8 else 0)) * 18 + 10 + (lane % 8)
                     for lane in range(16)])
    rows.append([lane % 8 for lane in range(16)])
    return np.array(rows, np.int32)


_CMAPS = _build_cmaps()
_ROW_MAP = 0
_ROW_PMSK = _NU
_ROW_CPOS = 2 * _NU
_ROW_OPOS = 2 * _NU + _NV
_ROW_CPAT = 2 * _NU + 2 * _NV


@jax.jit
def kernel(x, type_embedding):
    b, s = x.shape[0], x.shape[1]
    n = b * s
    xflat = x.reshape(n * _IN)
    tab = type_embedding.reshape(_NT * _ED)

    info = plsc.get_sparse_core_info()
    nw = info.num_cores * info.num_subcores
    rows_pw = n // nw
    chunk = 128
    while rows_pw % chunk != 0:
        chunk //= 2
    n_chunks = rows_pw // chunk

    mesh = plsc.VectorSubcoreMesh(core_axis_name="c", subcore_axis_name="s")

    @functools.partial(
        pl.kernel,
        mesh=mesh,
        compiler_params=pltpu.CompilerParams(needs_layout_passes=False),
        out_type=jax.ShapeDtypeStruct((n * _OUT,), jnp.float32),
        scratch_types=[
            pltpu.VMEM((_CMAPS.size,), jnp.int32),
            pltpu.VMEM((_NT * _ED,), jnp.float32),
            pltpu.VMEM((chunk * _IN,), jnp.float32),
            pltpu.VMEM((chunk * _OUT,), jnp.float32),
            pltpu.SemaphoreType.DMA,
            pltpu.SemaphoreType.DMA,
        ],
    )
    def k(x_hbm, tab_hbm, cm_hbm, out_hbm, cm_v, tab_v, in_v, out_v,
          sem_in, sem_out):
        wid = lax.axis_index("s") * info.num_cores + lax.axis_index("c")
        base = wid * rows_pw
        pltpu.sync_copy(cm_hbm, cm_v)
        pltpu.sync_copy(tab_hbm, tab_v)
        lanes = lax.iota(jnp.int32, 16)

        def chunk_body(ci, carry):
            r0 = base + ci * chunk
            pltpu.async_copy(
                x_hbm.at[pl.ds(r0 * _IN, chunk * _IN)], in_v, sem_in
            ).wait()

            for u in range(_NU):
                m = cm_v[pl.ds((_ROW_MAP + u) * 16, 16)]
                pm = cm_v[pl.ds((_ROW_PMSK + u) * 16, 16)] != 0
                oidx0 = lanes + u * 16

                @plsc.parallel_loop(0, chunk, unroll=8)
                def pbody(i, m=m, pm=pm, oidx0=oidx0):
                    vals = plsc.load_gather(in_v, [m + i * _IN])
                    plsc.store_scatter(
                        out_v, [oidx0 + i * _OUT], vals, mask=pm
                    )

            cpat = cm_v[pl.ds(_ROW_CPAT * 16, 16)]
            for v in range(_NV):
                cpos = cm_v[pl.ds((_ROW_CPOS + v) * 16, 16)]
                opos = cm_v[pl.ds((_ROW_OPOS + v) * 16, 16)]

                @plsc.parallel_loop(0, chunk, unroll=8)
                def ebody(i, cpos=cpos, opos=opos):
                    ch0 = plsc.load_gather(in_v, [cpos + i * _IN])
                    ei = ch0.astype(jnp.int32) * _ED + cpat
                    ev = plsc.load_gather(tab_v, [ei])
                    plsc.store_scatter(out_v, [opos + i * _OUT], ev)

            pltpu.async_copy(
                out_v, out_hbm.at[pl.ds(r0 * _OUT, chunk * _OUT)], sem_out
            ).wait()
            return carry

        lax.fori_loop(0, n_chunks, chunk_body, 0)

    out = k(xflat, tab, jnp.asarray(_CMAPS.reshape(-1)))
    return out.reshape(b, s, _OUT)


# speedup vs baseline: 1.0723x; 1.0160x over previous
import functools

import jax
import jax.numpy as jnp
import numpy as np
from jax import lax
from jax.experimental import pallas as pl
from jax.experimental.pallas import tpu as pltpu
from jax.experimental.pallas import tpu_sc as plsc

_MD = 11
_G = 24
_NT = 19
_ED = 8
_IN = _G * _MD
_OUT = _G * (_MD - 1 + _ED)
_NU = _OUT // 16
_NV = _G // 2


def _build_cmaps():
    rows = []
    masks = []
    for u in range(_NU):
        row = []
        mrow = []
        for lane in range(16):
            q = u * 16 + lane
            g, r = divmod(q, 18)
            row.append(g * _MD + 1 + r if r < 10 else g * _MD)
            mrow.append(1 if r < 10 else 0)
        rows.append(row)
        masks.append(mrow)
    rows.extend(masks)
    for v in range(_NV):
        rows.append([(2 * v + (1 if lane >= 8 else 0)) * _MD
                     for lane in range(16)])
    for v in range(_NV):
        rows.append([(2 * v + (1 if lane >= 8 else 0)) * 18 + 10 + (lane % 8)
                     for lane in range(16)])
    rows.append([lane % 8 for lane in range(16)])
    return np.array(rows, np.int32)


_CMAPS = _build_cmaps()
_ROW_MAP = 0
_ROW_PMSK = _NU
_ROW_CPOS = 2 * _NU
_ROW_OPOS = 2 * _NU + _NV
_ROW_CPAT = 2 * _NU + 2 * _NV

_CB = 2


@jax.jit
def kernel(x, type_embedding):
    b, s = x.shape[0], x.shape[1]

    info = plsc.get_sparse_core_info()
    nw = info.num_cores * info.num_subcores
    b_pw = b // nw
    n_chunks = b_pw // _CB

    mesh = plsc.VectorSubcoreMesh(core_axis_name="c", subcore_axis_name="s")

    @functools.partial(
        pl.kernel,
        mesh=mesh,
        compiler_params=pltpu.CompilerParams(needs_layout_passes=False),
        out_type=jax.ShapeDtypeStruct((b, s, _OUT), jnp.float32),
        scratch_types=[
            pltpu.VMEM((_CMAPS.size,), jnp.int32),
            pltpu.VMEM((_NT, _ED), jnp.float32),
            pltpu.VMEM((s, _IN), jnp.float32),
            pltpu.VMEM((s, _IN), jnp.float32),
            pltpu.VMEM((s, _OUT), jnp.float32),
            pltpu.VMEM((s, _OUT), jnp.float32),
            pltpu.SemaphoreType.DMA,
            pltpu.SemaphoreType.DMA,
        ],
    )
    def k(x_hbm, tab_hbm, cm_hbm, out_hbm, cm_v, tab_v, in_v0, in_v1,
          out_v0, out_v1, sem_in, sem_out):
        wid = lax.axis_index("s") * info.num_cores + lax.axis_index("c")
        base = wid * b_pw
        pltpu.sync_copy(cm_hbm, cm_v)
        pltpu.sync_copy(tab_hbm, tab_v)
        zeros16 = lax.iota(jnp.int32, 16) * 0
        lanes = lax.iota(jnp.int32, 16)

        def chunk_body(ci, carry):
            b0 = base + ci * _CB
            cin0 = pltpu.async_copy(x_hbm.at[b0], in_v0, sem_in)
            cin1 = pltpu.async_copy(x_hbm.at[b0 + 1], in_v1, sem_in)
            cin0.wait()
            cin1.wait()

            cpat = cm_v[pl.ds(_ROW_CPAT * 16, 16)]
            for in_v, out_v in ((in_v0, out_v0), (in_v1, out_v1)):
                for u in range(_NU):
                    m = cm_v[pl.ds((_ROW_MAP + u) * 16, 16)]
                    pm = cm_v[pl.ds((_ROW_PMSK + u) * 16, 16)] != 0
                    oidx0 = lanes + u * 16

                    @plsc.parallel_loop(0, s, unroll=5)
                    def pbody(i, m=m, pm=pm, oidx0=oidx0,
                              in_v=in_v, out_v=out_v):
                        spl = zeros16 + i
                        vals = plsc.load_gather(in_v, [spl, m])
                        plsc.store_scatter(out_v, [spl, oidx0], vals, mask=pm)

                for v in range(_NV):
                    cpos = cm_v[pl.ds((_ROW_CPOS + v) * 16, 16)]
                    opos = cm_v[pl.ds((_ROW_OPOS + v) * 16, 16)]

                    @plsc.parallel_loop(0, s, unroll=5)
                    def ebody(i, cpos=cpos, opos=opos,
                              in_v=in_v, out_v=out_v):
                        spl = zeros16 + i
                        ch0 = plsc.load_gather(in_v, [spl, cpos])
                        ei = ch0.astype(jnp.int32)
                        ev = plsc.load_gather(tab_v, [ei, cpat])
                        plsc.store_scatter(out_v, [spl, opos], ev)

            cout0 = pltpu.async_copy(out_v0, out_hbm.at[b0], sem_out)
            cout1 = pltpu.async_copy(out_v1, out_hbm.at[b0 + 1], sem_out)
            cout0.wait()
            cout1.wait()
            return carry

        lax.fori_loop(0, n_chunks, chunk_body, 0)

    out = k(x, type_embedding, jnp.asarray(_CMAPS.reshape(-1)))
    return out


# speedup vs baseline: 1.3792x; 1.2862x over previous
import functools

import jax
import jax.numpy as jnp
import numpy as np
from jax import lax
from jax.experimental import pallas as pl
from jax.experimental.pallas import tpu as pltpu
from jax.experimental.pallas import tpu_sc as plsc

_MD = 11
_G = 24
_NT = 19
_ED = 8
_IN = _G * _MD
_OUT = _G * (_MD - 1 + _ED)
_NU = _OUT // 16
_NV = _G // 2


def _build_cmaps():
    rows = []
    masks = []
    for u in range(_NU):
        row = []
        mrow = []
        for lane in range(16):
            q = u * 16 + lane
            g, r = divmod(q, 18)
            row.append(g * _MD + 1 + r if r < 10 else g * _MD)
            mrow.append(1 if r < 10 else 0)
        rows.append(row)
        masks.append(mrow)
    rows.extend(masks)
    for v in range(_NV):
        rows.append([(2 * v + (1 if lane >= 8 else 0)) * _MD
                     for lane in range(16)])
    for v in range(_NV):
        rows.append([(2 * v + (1 if lane >= 8 else 0)) * 18 + 10 + (lane % 8)
                     for lane in range(16)])
    rows.append([lane % 8 for lane in range(16)])
    return np.array(rows, np.int32)


_CMAPS = _build_cmaps()
_ROW_MAP = 0
_ROW_PMSK = _NU
_ROW_CPOS = 2 * _NU
_ROW_OPOS = 2 * _NU + _NV
_ROW_CPAT = 2 * _NU + 2 * _NV

_CB = 2


@jax.jit
def kernel(x, type_embedding):
    b, s = x.shape[0], x.shape[1]

    info = plsc.get_sparse_core_info()
    nw = info.num_cores * info.num_subcores
    b_pw = b // nw
    n_chunks = b_pw // _CB

    mesh = plsc.VectorSubcoreMesh(core_axis_name="c", subcore_axis_name="s")

    @functools.partial(
        pl.kernel,
        mesh=mesh,
        compiler_params=pltpu.CompilerParams(needs_layout_passes=False),
        out_type=jax.ShapeDtypeStruct((b, s, _OUT), jnp.float32),
        scratch_types=[
            pltpu.VMEM((_CMAPS.size,), jnp.int32),
            pltpu.VMEM((_NT, _ED), jnp.float32),
            pltpu.VMEM((s, _IN), jnp.float32),
            pltpu.VMEM((s, _IN), jnp.float32),
            pltpu.VMEM((s, _OUT), jnp.float32),
            pltpu.VMEM((s, _OUT), jnp.float32),
            pltpu.SemaphoreType.DMA,
            pltpu.SemaphoreType.DMA,
        ],
    )
    def k(x_hbm, tab_hbm, cm_hbm, out_hbm, cm_v, tab_v, in_v0, in_v1,
          out_v0, out_v1, sem_in, sem_out):
        wid = lax.axis_index("s") * info.num_cores + lax.axis_index("c")
        base = wid * b_pw
        pltpu.sync_copy(cm_hbm, cm_v)
        pltpu.sync_copy(tab_hbm, tab_v)
        zeros16 = lax.iota(jnp.int32, 16) * 0
        lanes = lax.iota(jnp.int32, 16)

        def chunk_body(ci, carry):
            b0 = base + ci * _CB
            cin0 = pltpu.async_copy(x_hbm.at[b0], in_v0, sem_in)
            cin1 = pltpu.async_copy(x_hbm.at[b0 + 1], in_v1, sem_in)
            cin0.wait()
            cin1.wait()

            @pl.when(ci > 0)
            def _drain():
                pltpu.make_async_copy(
                    out_v0, out_hbm.at[b0 - _CB], sem_out
                ).wait()
                pltpu.make_async_copy(
                    out_v1, out_hbm.at[b0 - _CB + 1], sem_out
                ).wait()

            cpat = cm_v[pl.ds(_ROW_CPAT * 16, 16)]
            for u in range(_NU):
                m = cm_v[pl.ds((_ROW_MAP + u) * 16, 16)]
                pm = cm_v[pl.ds((_ROW_PMSK + u) * 16, 16)] != 0
                oidx0 = lanes + u * 16

                @plsc.parallel_loop(0, s, unroll=5)
                def pbody(i, m=m, pm=pm, oidx0=oidx0):
                    spl = zeros16 + i
                    vals0 = plsc.load_gather(in_v0, [spl, m])
                    plsc.store_scatter(out_v0, [spl, oidx0], vals0, mask=pm)
                    vals1 = plsc.load_gather(in_v1, [spl, m])
                    plsc.store_scatter(out_v1, [spl, oidx0], vals1, mask=pm)

            for v in range(_NV):
                cpos = cm_v[pl.ds((_ROW_CPOS + v) * 16, 16)]
                opos = cm_v[pl.ds((_ROW_OPOS + v) * 16, 16)]

                @plsc.parallel_loop(0, s, unroll=5)
                def ebody(i, cpos=cpos, opos=opos):
                    spl = zeros16 + i
                    ch00 = plsc.load_gather(in_v0, [spl, cpos])
                    ev0 = plsc.load_gather(tab_v, [ch00.astype(jnp.int32),
                                                   cpat])
                    plsc.store_scatter(out_v0, [spl, opos], ev0)
                    ch01 = plsc.load_gather(in_v1, [spl, cpos])
                    ev1 = plsc.load_gather(tab_v, [ch01.astype(jnp.int32),
                                                   cpat])
                    plsc.store_scatter(out_v1, [spl, opos], ev1)

            pltpu.async_copy(out_v0, out_hbm.at[b0], sem_out)
            pltpu.async_copy(out_v1, out_hbm.at[b0 + 1], sem_out)
            return carry

        lax.fori_loop(0, n_chunks, chunk_body, 0)
        lastb = base + (n_chunks - 1) * _CB
        pltpu.make_async_copy(out_v0, out_hbm.at[lastb], sem_out).wait()
        pltpu.make_async_copy(out_v1, out_hbm.at[lastb + 1], sem_out).wait()

    out = k(x, type_embedding, jnp.asarray(_CMAPS.reshape(-1)))
    return out
